# baseline (speedup 1.0000x reference)
import jax
import jax.numpy as jnp
from jax import lax
from jax.experimental import pallas as pl
from jax.experimental.pallas import tpu as pltpu
from jax.experimental.pallas import tpu_sc as plsc

N_NODES = 10000
N_EDGES = 160000
D = 256
DH = 128

NC = 2
NS = 16
E_PAD = 163840
EDGES_PER_TILE = E_PAD // NS
CHUNK = 80
N_CHUNKS = EDGES_PER_TILE // CHUNK
N_PAD = 10240
ROWS_PER_TILE = N_PAD // NS

NBUF = 5
PREFETCH = NBUF - 1
IRING = PREFETCH + 2
SRING = 2


def _sc_body(xcat_ref, ed_ref, out_ref,
             edr, rows, sbuf, acc, isem, gsem, ssem):
    c = lax.axis_index("c")
    s = lax.axis_index("s")

    chunk_base = s * N_CHUNKS

    def idx_start(ch):
        i = lax.rem(ch, IRING)
        pltpu.make_async_copy(ed_ref.at[c, chunk_base + ch], edr.at[i],
                              isem.at[i]).start()

    def idx_wait(ch):
        i = lax.rem(ch, IRING)
        pltpu.make_async_copy(ed_ref.at[c, chunk_base + ch], edr.at[i],
                              isem.at[i]).wait()

    def gather_start(ch, b):
        i = lax.rem(ch, IRING)
        pltpu.make_async_copy(xcat_ref.at[edr.at[i, pl.ds(0, CHUNK)]],
                              rows.at[pl.ds(b * CHUNK, CHUNK)],
                              gsem.at[b]).start()

    def gather_wait(ch, b):
        i = lax.rem(ch, IRING)
        pltpu.make_async_copy(xcat_ref.at[edr.at[i, pl.ds(0, CHUNK)]],
                              rows.at[pl.ds(b * CHUNK, CHUNK)],
                              gsem.at[b]).wait()

    def scatter_start(ch, sb):
        i = lax.rem(ch, IRING)
        pltpu.make_async_copy(sbuf.at[pl.ds(sb * CHUNK, CHUNK)],
                              acc.at[edr.at[i, pl.ds(CHUNK, CHUNK)]], ssem.at[sb]).start(add=True)

    def scatter_wait(ch, sb):
        i = lax.rem(ch, IRING)
        pltpu.make_async_copy(sbuf.at[pl.ds(sb * CHUNK, CHUNK)],
                              acc.at[edr.at[i, pl.ds(CHUNK, CHUNK)]], ssem.at[sb]).wait()

    def scale(ch, b, sb):
        i = lax.rem(ch, IRING)

        def scale_i(g, inner):
            av = lax.bitcast_convert_type(edr[i, pl.ds(2 * CHUNK + g * 16, 16)], jnp.float32)
            for l in range(16):
                a = av[l]
                e = g * 16 + l
                er = b * CHUNK + e
                es = sb * CHUNK + e
                for k in range(DH // 32):
                    v = rows[er, pl.ds(k * 16, 16)]
                    lo = lax.bitcast_convert_type(v << 16, jnp.float32)
                    hi = lax.bitcast_convert_type(v & jnp.int32(-65536), jnp.float32)
                    sbuf[es, pl.ds(k * 32, 16)] = lo * a
                    sbuf[es, pl.ds(k * 32 + 16, 16)] = hi * a
            return inner
        lax.fori_loop(0, CHUNK // 16, scale_i, 0)

    for pch in range(PREFETCH + 1):
        idx_start(jnp.int32(pch))

    def zero_row(r, carry):
        for k in range(DH // 16):
            sbuf[r, pl.ds(k * 16, 16)] = jnp.zeros((16,), jnp.float32)
        return carry
    lax.fori_loop(0, CHUNK, zero_row, 0)

    base_n = s * ROWS_PER_TILE
    for z in range(ROWS_PER_TILE // CHUNK):
        pltpu.sync_copy(sbuf.at[pl.ds(0, CHUNK)],
                        acc.at[pl.ds(base_n + z * CHUNK, CHUNK)])
    plsc.subcore_barrier()

    for pch in range(PREFETCH):
        idx_wait(jnp.int32(pch))
        gather_start(jnp.int32(pch), pch % NBUF)

    def body(ch, carry):
        b = lax.rem(ch, NBUF)
        sb = lax.rem(ch, SRING)
        gather_wait(ch, b)
        scale(ch, b, sb)
        scatter_start(ch, sb)

        @pl.when(ch >= 1)
        def _():
            scatter_wait(ch - 1, lax.rem(ch + 1, SRING))

        nb = lax.rem(ch + PREFETCH, NBUF)

        @pl.when(ch + PREFETCH < N_CHUNKS)
        def _():
            idx_wait(ch + PREFETCH)
            gather_start(ch + PREFETCH, nb)

        @pl.when(ch + PREFETCH + 1 < N_CHUNKS)
        def _():
            idx_start(ch + PREFETCH + 1)
        return carry
    lax.fori_loop(0, N_CHUNKS, body, 0)

    scatter_wait(jnp.int32(N_CHUNKS - 1), (N_CHUNKS - 1) % SRING)

    plsc.subcore_barrier()
    pltpu.sync_copy(acc.at[pl.ds(base_n, ROWS_PER_TILE)],
                    out_ref.at[c, pl.ds(base_n, ROWS_PER_TILE)])


def _sc_segment_sum(xcat, edata):
    mesh = plsc.VectorSubcoreMesh(core_axis_name="c", subcore_axis_name="s")
    return pl.kernel(
        _sc_body,
        out_type=jax.ShapeDtypeStruct((NC, N_PAD, DH), jnp.float32),
        mesh=mesh,
        compiler_params=pltpu.CompilerParams(use_tc_tiling_on_sc=False),
        scratch_types=[
            pltpu.VMEM((IRING, 3 * CHUNK), jnp.int32),
            pltpu.VMEM((NBUF * CHUNK, DH // 2), jnp.int32),
            pltpu.VMEM((SRING * CHUNK, DH), jnp.float32),
            pltpu.VMEM_SHARED((N_PAD, DH), jnp.float32),
            pltpu.SemaphoreType.DMA((IRING,)),
            pltpu.SemaphoreType.DMA((NBUF,)),
            pltpu.SemaphoreType.DMA((SRING,)),
        ],
    )(xcat, edata)


def _tc_body(x_ref, nhlo_ref, nhhi_ref, w1_ref, w2_ref, b1_ref, b2_ref, o_ref):
    x = x_ref[...]
    nh = jnp.concatenate([nhlo_ref[0], nhhi_ref[0]], axis=1)
    dn = (((1,), (1,)), ((), ()))
    y1 = lax.dot_general(x + nh, w1_ref[...], dn,
                         preferred_element_type=jnp.float32) + b1_ref[...]
    y2 = lax.dot_general(x * nh, w2_ref[...], dn,
                         preferred_element_type=jnp.float32) + b2_ref[...]
    y1 = jnp.where(y1 > 0, y1, 0.01 * y1)
    y2 = jnp.where(y2 > 0, y2, 0.01 * y2)
    o_ref[...] = y1 + y2


def _tc_biinteract(x, nh2, W1, W2, b1, b2):
    R = 400
    grid = (N_NODES // R,)
    return pl.pallas_call(
        _tc_body,
        grid=grid,
        in_specs=[
            pl.BlockSpec((R, D), lambda i: (i, 0)),
            pl.BlockSpec((1, R, DH), lambda i: (0, i, 0)),
            pl.BlockSpec((1, R, DH), lambda i: (1, i, 0)),
            pl.BlockSpec((D, D), lambda i: (0, 0)),
            pl.BlockSpec((D, D), lambda i: (0, 0)),
            pl.BlockSpec((1, D), lambda i: (0, 0)),
            pl.BlockSpec((1, D), lambda i: (0, 0)),
        ],
        out_specs=pl.BlockSpec((R, D), lambda i: (i, 0)),
        out_shape=jax.ShapeDtypeStruct((N_NODES, D), jnp.float32),
    )(x, nh2, nh2, W1, W2, b1, b2)


def kernel(entity_embed, edge_index, edge_att, W1, b1, W2, b2):
    x = entity_embed
    src = edge_index[0].astype(jnp.int32)
    dst = edge_index[1].astype(jnp.int32)
    att = edge_att.astype(jnp.float32)

    pad = E_PAD - N_EDGES
    src_p = jnp.concatenate([src, jnp.zeros((pad,), jnp.int32)])
    dst_p = jnp.concatenate([dst, jnp.zeros((pad,), jnp.int32)])
    att_p = jnp.concatenate([att, jnp.zeros((pad,), jnp.float32)])

    src2 = jnp.stack([src_p, src_p + N_NODES])
    src3 = src2.reshape(NC, E_PAD // CHUNK, CHUNK)
    dst3 = jnp.broadcast_to(dst_p.reshape(1, E_PAD // CHUNK, CHUNK),
                            (NC, E_PAD // CHUNK, CHUNK))
    att3 = jnp.broadcast_to(
        lax.bitcast_convert_type(att_p, jnp.int32).reshape(1, E_PAD // CHUNK, CHUNK),
        (NC, E_PAD // CHUNK, CHUNK))
    edata = jnp.concatenate([src3, dst3, att3], axis=-1)

    xcat = jnp.concatenate([x[:, :DH], x[:, DH:]], axis=0).astype(jnp.bfloat16)
    xperm = xcat.reshape(2 * N_NODES, DH // 32, 2, 16).transpose(0, 1, 3, 2)
    xperm = lax.bitcast_convert_type(xperm, jnp.int32)
    xperm = xperm.reshape(2 * N_NODES, DH // 2)

    nh2 = _sc_segment_sum(xperm, edata)

    return _tc_biinteract(x, nh2, W1, W2,
                          b1.reshape(1, D), b2.reshape(1, D))


# speedup vs baseline: 1.0153x; 1.0153x over previous
import jax
import jax.numpy as jnp
from jax import lax
from jax.experimental import pallas as pl
from jax.experimental.pallas import tpu as pltpu
from jax.experimental.pallas import tpu_sc as plsc

N_NODES = 10000
N_EDGES = 160000
D = 256
DH = 128

NC = 2
NS = 16
E_PAD = 163840
EDGES_PER_TILE = E_PAD // NS
CHUNK = 80
N_CHUNKS = EDGES_PER_TILE // CHUNK
N_PAD = 10240
ROWS_PER_TILE = N_PAD // NS

NBUF = 5
PREFETCH = NBUF - 1
IRING = PREFETCH + 2
SRING = 2


def _sc_body(xcat_ref, src_ref, dst_ref, att_ref, out_ref,
             sidxr, didxr, attr, rows, sbuf, acc, isem, gsem, ssem):
    c = lax.axis_index("c")
    s = lax.axis_index("s")

    chunk_base = s * N_CHUNKS

    def idx_start(ch):
        i = lax.rem(ch, IRING)
        pltpu.make_async_copy(src_ref.at[c, chunk_base + ch], sidxr.at[i],
                              isem.at[i]).start()
        pltpu.make_async_copy(dst_ref.at[chunk_base + ch], didxr.at[i],
                              isem.at[i]).start()
        pltpu.make_async_copy(att_ref.at[chunk_base + ch], attr.at[i],
                              isem.at[i]).start()

    def idx_wait(ch):
        i = lax.rem(ch, IRING)
        pltpu.make_async_copy(src_ref.at[c, chunk_base + ch], sidxr.at[i],
                              isem.at[i]).wait()
        pltpu.make_async_copy(dst_ref.at[chunk_base + ch], didxr.at[i],
                              isem.at[i]).wait()
        pltpu.make_async_copy(att_ref.at[chunk_base + ch], attr.at[i],
                              isem.at[i]).wait()

    def gather_start(ch, b):
        i = lax.rem(ch, IRING)
        pltpu.make_async_copy(xcat_ref.at[sidxr.at[i]],
                              rows.at[pl.ds(b * CHUNK, CHUNK)],
                              gsem.at[b]).start()

    def gather_wait(ch, b):
        i = lax.rem(ch, IRING)
        pltpu.make_async_copy(xcat_ref.at[sidxr.at[i]],
                              rows.at[pl.ds(b * CHUNK, CHUNK)],
                              gsem.at[b]).wait()

    def scatter_start(ch, sb):
        i = lax.rem(ch, IRING)
        pltpu.make_async_copy(sbuf.at[pl.ds(sb * CHUNK, CHUNK)],
                              acc.at[didxr.at[i]], ssem.at[sb]).start(add=True)

    def scatter_wait(ch, sb):
        i = lax.rem(ch, IRING)
        pltpu.make_async_copy(sbuf.at[pl.ds(sb * CHUNK, CHUNK)],
                              acc.at[didxr.at[i]], ssem.at[sb]).wait()

    def scale(ch, b, sb):
        i = lax.rem(ch, IRING)

        def scale_i(g, inner):
            av = attr[i, pl.ds(g * 16, 16)]
            for l in range(16):
                a = av[l]
                e = g * 16 + l
                er = b * CHUNK + e
                es = sb * CHUNK + e
                for k in range(DH // 32):
                    v = rows[er, pl.ds(k * 16, 16)]
                    lo = lax.bitcast_convert_type(v << 16, jnp.float32)
                    hi = lax.bitcast_convert_type(v & jnp.int32(-65536), jnp.float32)
                    sbuf[es, pl.ds(k * 32, 16)] = lo * a
                    sbuf[es, pl.ds(k * 32 + 16, 16)] = hi * a
            return inner
        lax.fori_loop(0, CHUNK // 16, scale_i, 0)

    for pch in range(PREFETCH + 1):
        idx_start(jnp.int32(pch))

    def zero_row(r, carry):
        for k in range(DH // 16):
            sbuf[r, pl.ds(k * 16, 16)] = jnp.zeros((16,), jnp.float32)
        return carry
    lax.fori_loop(0, CHUNK, zero_row, 0)

    base_n = s * ROWS_PER_TILE
    for z in range(ROWS_PER_TILE // CHUNK):
        pltpu.sync_copy(sbuf.at[pl.ds(0, CHUNK)],
                        acc.at[pl.ds(base_n + z * CHUNK, CHUNK)])
    plsc.subcore_barrier()

    for pch in range(PREFETCH):
        idx_wait(jnp.int32(pch))
        gather_start(jnp.int32(pch), pch % NBUF)

    def body(ch, carry):
        b = lax.rem(ch, NBUF)
        sb = lax.rem(ch, SRING)
        gather_wait(ch, b)
        scale(ch, b, sb)
        scatter_start(ch, sb)

        @pl.when(ch >= 1)
        def _():
            scatter_wait(ch - 1, lax.rem(ch + 1, SRING))

        nb = lax.rem(ch + PREFETCH, NBUF)

        @pl.when(ch + PREFETCH < N_CHUNKS)
        def _():
            idx_wait(ch + PREFETCH)
            gather_start(ch + PREFETCH, nb)

        @pl.when(ch + PREFETCH + 1 < N_CHUNKS)
        def _():
            idx_start(ch + PREFETCH + 1)
        return carry
    lax.fori_loop(0, N_CHUNKS, body, 0)

    scatter_wait(jnp.int32(N_CHUNKS - 1), (N_CHUNKS - 1) % SRING)

    plsc.subcore_barrier()
    pltpu.sync_copy(acc.at[pl.ds(base_n, ROWS_PER_TILE)],
                    out_ref.at[c, pl.ds(base_n, ROWS_PER_TILE)])


def _sc_segment_sum(xcat, src3, dst3, att3):
    mesh = plsc.VectorSubcoreMesh(core_axis_name="c", subcore_axis_name="s")
    return pl.kernel(
        _sc_body,
        out_type=jax.ShapeDtypeStruct((NC, N_PAD, DH), jnp.float32),
        mesh=mesh,
        compiler_params=pltpu.CompilerParams(use_tc_tiling_on_sc=False),
        scratch_types=[
            pltpu.VMEM((IRING, CHUNK), jnp.int32),
            pltpu.VMEM((IRING, CHUNK), jnp.int32),
            pltpu.VMEM((IRING, CHUNK), jnp.float32),
            pltpu.VMEM((NBUF * CHUNK, DH // 2), jnp.int32),
            pltpu.VMEM((SRING * CHUNK, DH), jnp.float32),
            pltpu.VMEM_SHARED((N_PAD, DH), jnp.float32),
            pltpu.SemaphoreType.DMA((IRING,)),
            pltpu.SemaphoreType.DMA((NBUF,)),
            pltpu.SemaphoreType.DMA((SRING,)),
        ],
    )(xcat, src3, dst3, att3)


def _tc_body(x_ref, nhlo_ref, nhhi_ref, w1_ref, w2_ref, b1_ref, b2_ref, o_ref):
    x = x_ref[...]
    nh = jnp.concatenate([nhlo_ref[0], nhhi_ref[0]], axis=1)
    dn = (((1,), (1,)), ((), ()))
    y1 = lax.dot_general(x + nh, w1_ref[...], dn,
                         preferred_element_type=jnp.float32) + b1_ref[...]
    y2 = lax.dot_general(x * nh, w2_ref[...], dn,
                         preferred_element_type=jnp.float32) + b2_ref[...]
    y1 = jnp.where(y1 > 0, y1, 0.01 * y1)
    y2 = jnp.where(y2 > 0, y2, 0.01 * y2)
    o_ref[...] = y1 + y2


def _tc_biinteract(x, nh2, W1, W2, b1, b2):
    R = 400
    grid = (N_NODES // R,)
    return pl.pallas_call(
        _tc_body,
        grid=grid,
        in_specs=[
            pl.BlockSpec((R, D), lambda i: (i, 0)),
            pl.BlockSpec((1, R, DH), lambda i: (0, i, 0)),
            pl.BlockSpec((1, R, DH), lambda i: (1, i, 0)),
            pl.BlockSpec((D, D), lambda i: (0, 0)),
            pl.BlockSpec((D, D), lambda i: (0, 0)),
            pl.BlockSpec((1, D), lambda i: (0, 0)),
            pl.BlockSpec((1, D), lambda i: (0, 0)),
        ],
        out_specs=pl.BlockSpec((R, D), lambda i: (i, 0)),
        out_shape=jax.ShapeDtypeStruct((N_NODES, D), jnp.float32),
    )(x, nh2, nh2, W1, W2, b1, b2)


def kernel(entity_embed, edge_index, edge_att, W1, b1, W2, b2):
    x = entity_embed
    src = edge_index[0].astype(jnp.int32)
    dst = edge_index[1].astype(jnp.int32)
    att = edge_att.astype(jnp.float32)

    pad = E_PAD - N_EDGES
    src_p = jnp.concatenate([src, jnp.zeros((pad,), jnp.int32)])
    dst_p = jnp.concatenate([dst, jnp.zeros((pad,), jnp.int32)])
    att_p = jnp.concatenate([att, jnp.zeros((pad,), jnp.float32)])

    src2 = jnp.stack([src_p, src_p + N_NODES])
    src3 = src2.reshape(NC, E_PAD // CHUNK, CHUNK)
    dst3 = dst_p.reshape(E_PAD // CHUNK, CHUNK)
    att3 = att_p.reshape(E_PAD // CHUNK, CHUNK)

    xcat = jnp.concatenate([x[:, :DH], x[:, DH:]], axis=0).astype(jnp.bfloat16)
    xperm = xcat.reshape(2 * N_NODES, DH // 32, 2, 16).transpose(0, 1, 3, 2)
    xperm = lax.bitcast_convert_type(xperm, jnp.int32)
    xperm = xperm.reshape(2 * N_NODES, DH // 2)

    nh2 = _sc_segment_sum(xperm, src3, dst3, att3)

    return _tc_biinteract(x, nh2, W1, W2,
                          b1.reshape(1, D), b2.reshape(1, D))
